# baseline (speedup 1.0000x reference)
import jax
import jax.numpy as jnp
from jax import lax
from jax.experimental import pallas as pl
from jax.experimental.pallas import tpu as pltpu

_MB = 1024


def _argmax_body(x_ref, o_ref, vmax_ref, vidx_ref):
    mi = pl.program_id(1)
    nmb = pl.num_programs(1)
    blk = x_ref[0]
    mb = blk.shape[0]
    mx = jnp.max(blk, axis=0)
    iota = lax.broadcasted_iota(jnp.int32, blk.shape, 0)
    idx = jnp.min(jnp.where(blk == mx[None, :], iota, mb), axis=0) + mi * mb

    @pl.when(mi == 0)
    def _init():
        vmax_ref[0] = mx
        vidx_ref[0] = idx

    @pl.when(mi > 0)
    def _combine():
        better = mx > vmax_ref[0]
        vmax_ref[0] = jnp.where(better, mx, vmax_ref[0])
        vidx_ref[0] = jnp.where(better, idx, vidx_ref[0])

    @pl.when(mi == nmb - 1)
    def _emit():
        o_ref[0, 0] = vidx_ref[0]


def kernel(x):
    B, M, N = x.shape
    out = pl.pallas_call(
        _argmax_body,
        grid=(B, M // _MB),
        in_specs=[pl.BlockSpec((1, _MB, N), lambda b, m: (b, m, 0))],
        out_specs=pl.BlockSpec((1, 1, N), lambda b, m: (b, 0, 0)),
        out_shape=jax.ShapeDtypeStruct((B, 1, N), jnp.int32),
        scratch_shapes=[
            pltpu.VMEM((1, N), jnp.float32),
            pltpu.VMEM((1, N), jnp.int32),
        ],
    )(x)
    return out.reshape(B, N)


# speedup vs baseline: 1.2347x; 1.2347x over previous
import jax
import jax.numpy as jnp
from jax import lax
from jax.experimental import pallas as pl
from jax.experimental.pallas import tpu as pltpu


def _half_argmax(blk):
    m = blk.shape[0]
    mx = jnp.max(blk, axis=0)
    iota = lax.broadcasted_iota(jnp.int32, blk.shape, 0)
    idx = jnp.min(jnp.where(blk == mx[None, :], iota, m), axis=0)
    return mx, idx


def _argmax_body(x1_ref, x2_ref, o_ref):
    m1 = x1_ref.shape[1]
    mx1, idx1 = _half_argmax(x1_ref[0])
    mx2, idx2 = _half_argmax(x2_ref[0])
    first_low = mx1 >= mx2
    o_ref[0, 0] = jnp.where(first_low, idx1, idx2 + m1)


def kernel(x):
    B, M, N = x.shape
    MH = M // 2
    out = pl.pallas_call(
        _argmax_body,
        grid=(B,),
        in_specs=[
            pl.BlockSpec((1, MH, N), lambda b: (b, 0, 0)),
            pl.BlockSpec((1, MH, N), lambda b: (b, 1, 0)),
        ],
        out_specs=pl.BlockSpec((1, 1, N), lambda b: (b, 0, 0)),
        out_shape=jax.ShapeDtypeStruct((B, 1, N), jnp.int32),
    )(x, x)
    return out.reshape(B, N)
